# Initial kernel scaffold; baseline (speedup 1.0000x reference)
import jax
import jax.numpy as jnp
from jax.experimental import pallas as pl


def kernel(x, seg, tok_table, pos_table, seg_table, gamma, beta):
    raise NotImplementedError("write your pallas kernel here")


# speedup vs baseline: 8.2300x; 8.2300x over previous
import jax
import jax.numpy as jnp
from jax.experimental import pallas as pl

B, L, D = 16384, 30, 768
BB = 128


def _body(x_ref, seg_ref, tok_ref, pos_ref, seg_t_ref, gamma_ref, beta_ref, out_ref):
    x = x_ref[...][:, :, None]
    seg = seg_ref[...][:, :, None]
    seq_len = x_ref.shape[1]

    e = jnp.broadcast_to(tok_ref[0, :][None, None, :], (x.shape[0], seq_len, D))
    for v in range(1, 4):
        e = jnp.where(x == v, tok_ref[v, :][None, None, :], e)
    e = e + jnp.where(seg == 0, seg_t_ref[0, :][None, None, :],
                      seg_t_ref[1, :][None, None, :])
    e = e + pos_ref[0:seq_len, :][None, :, :]

    mean = jnp.mean(e, axis=-1, keepdims=True)
    c = e - mean
    var = jnp.mean(c * c, axis=-1, keepdims=True)
    inv = jax.lax.rsqrt(var + 1e-5)
    out_ref[...] = c * inv * gamma_ref[0, :][None, None, :] + beta_ref[0, :][None, None, :]


def kernel(x, seg, tok_table, pos_table, seg_table, gamma, beta):
    b, seq_len = x.shape
    d = tok_table.shape[1]
    gamma2 = gamma.reshape(1, d)
    beta2 = beta.reshape(1, d)
    grid = (b // BB,)
    return pl.pallas_call(
        _body,
        grid=grid,
        in_specs=[
            pl.BlockSpec((BB, seq_len), lambda i: (i, 0)),
            pl.BlockSpec((BB, seq_len), lambda i: (i, 0)),
            pl.BlockSpec(tok_table.shape, lambda i: (0, 0)),
            pl.BlockSpec(pos_table.shape, lambda i: (0, 0)),
            pl.BlockSpec(seg_table.shape, lambda i: (0, 0)),
            pl.BlockSpec((1, d), lambda i: (0, 0)),
            pl.BlockSpec((1, d), lambda i: (0, 0)),
        ],
        out_specs=pl.BlockSpec((BB, seq_len, d), lambda i: (i, 0, 0)),
        out_shape=jax.ShapeDtypeStruct((b, seq_len, d), jnp.float32),
    )(x, seg, tok_table, pos_table, seg_table, gamma2, beta2)
